# baseline (speedup 1.0000x reference)
import functools

import jax
import jax.numpy as jnp
from jax import lax
from jax.experimental import pallas as pl
from jax.experimental.pallas import tpu as pltpu
from jax.experimental.pallas import tpu_sc as plsc

L = 200
B = 4096
EMB = 64
NCLS = 50
VOCAB = 100000
NP = 64

NC = 2
NS = 16
NW = NC * NS
BPW = B // NW

K = 24

VB = 512
PGRID = (VOCAB + VB - 1) // VB


def _proj_body(tT_ref, ws_ref, bs_ref, e_ref, o_ref):
    p = lax.dot_general(
        tT_ref[...], ws_ref[...],
        (((0,), (0,)), ((), ())),
        preferred_element_type=jnp.float32,
    )
    p = p + bs_ref[0:1, :]
    left = lax.dot_general(
        e_ref[0], p, (((1,), (0,)), ((), ())),
        preferred_element_type=jnp.float32,
    )
    right = lax.dot_general(
        e_ref[1], p, (((1,), (0,)), ((), ())),
        preferred_element_type=jnp.float32,
    )
    o_ref[...] = jnp.concatenate([left, right], axis=1)


def _project(tableT, Ws, bs, Esel):
    return pl.pallas_call(
        _proj_body,
        grid=(PGRID,),
        in_specs=[
            pl.BlockSpec((EMB, VB), lambda j: (0, j)),
            pl.BlockSpec((EMB, NP), lambda j: (0, 0)),
            pl.BlockSpec((8, NP), lambda j: (0, 0)),
            pl.BlockSpec((2, VB // 2, VB), lambda j: (0, 0, 0)),
        ],
        out_specs=pl.BlockSpec((VB // 2, 2 * NP), lambda j: (j, 0)),
        out_shape=jax.ShapeDtypeStruct((VOCAB // 2, 2 * NP), jnp.float32),
    )(tableT, Ws, bs, Esel)


def _sc_pool_body(docs_hbm, table_hbm, out_hbm, slab_v, acc_v, sem):
    wid = lax.axis_index("s") * NC + lax.axis_index("c")
    base = wid * BPW

    pltpu.sync_copy(docs_hbm.at[:, pl.ds(base, BPW)], slab_v)

    zeros = jnp.zeros((16,), jnp.float32)

    def zero_body(bb, c2):
        for e in range(NP // 16):
            acc_v[bb, pl.ds(e * 16, 16)] = zeros
        return c2

    lax.fori_loop(0, BPW, zero_body, 0, unroll=8)

    cps = []
    for l in range(L):
        cps.append(
            pltpu.async_copy(
                table_hbm.at[slab_v.at[l]], acc_v, sem, add=True
            )
        )
        if l >= K:
            cps[l - K].wait()
    for l in range(L - K, L):
        cps[l].wait()

    pltpu.sync_copy(acc_v, out_hbm.at[pl.ds(base, BPW)])


def _sc_pool(docs, ptable):
    mesh = plsc.VectorSubcoreMesh(core_axis_name="c", subcore_axis_name="s")
    f = pl.kernel(
        _sc_pool_body,
        out_type=jax.ShapeDtypeStruct((B, NP), jnp.float32),
        mesh=mesh,
        scratch_types=[
            pltpu.VMEM((L, BPW), jnp.int32),
            pltpu.VMEM((BPW, NP), jnp.float32),
            pltpu.SemaphoreType.DMA,
        ],
        compiler_params=pltpu.CompilerParams(use_tc_tiling_on_sc=False),
    )
    return f(docs, ptable)


@jax.jit
def _run(docs, table, W, b):
    tableT = jnp.transpose(table)
    Ws = jnp.zeros((EMB, NP), jnp.float32).at[:, :NCLS].set(W.T) * (1.0 / L)
    bs = jnp.broadcast_to(
        jnp.zeros((NP,), jnp.float32).at[:NCLS].set(b) * (1.0 / L), (8, NP)
    )
    half = VB // 2
    r_ids = lax.broadcasted_iota(jnp.int32, (half, VB), 0)
    s_ids = lax.broadcasted_iota(jnp.int32, (half, VB), 1)
    e_even = (s_ids == 2 * r_ids).astype(jnp.float32)
    e_odd = (s_ids == 2 * r_ids + 1).astype(jnp.float32)
    Esel = jnp.stack([e_even, e_odd], axis=0)

    p2 = _project(tableT, Ws, bs, Esel)
    ptable = jnp.reshape(p2, (VOCAB, NP))
    scores = _sc_pool(docs, ptable)
    return scores[:, :NCLS]


def kernel(docs, table, W, b):
    return _run(docs, table, W, b)


# speedup vs baseline: 1.4736x; 1.4736x over previous
import functools

import jax
import jax.numpy as jnp
from jax import lax
from jax.experimental import pallas as pl
from jax.experimental.pallas import tpu as pltpu
from jax.experimental.pallas import tpu_sc as plsc

L = 200
B = 4096
EMB = 64
NCLS = 50
VOCAB = 100000
NP = 64

NC = 2
NS = 16
NW = NC * NS
BPW = B // NW

K = 24

VB = 1024
PGRID = (VOCAB + VB - 1) // VB
VP = PGRID * VB


def _proj_body(tT_ref, ws_ref, bs_ref, o_ref):
    p = lax.dot_general(
        tT_ref[...], ws_ref[...],
        (((0,), (0,)), ((), ())),
        preferred_element_type=jnp.float32,
    )
    p = p + bs_ref[0:1, :]
    half = VB // 2
    o_ref[...] = jnp.concatenate([p[:half, :], p[half:, :]], axis=1)


def _project(tableT, Ws, bs):
    return pl.pallas_call(
        _proj_body,
        grid=(PGRID,),
        in_specs=[
            pl.BlockSpec((EMB, VB), lambda j: (0, j)),
            pl.BlockSpec((EMB, NP), lambda j: (0, 0)),
            pl.BlockSpec((8, NP), lambda j: (0, 0)),
        ],
        out_specs=pl.BlockSpec((VB // 2, 2 * NP), lambda j: (j, 0)),
        out_shape=jax.ShapeDtypeStruct((VP // 2, 2 * NP), jnp.float32),
    )(tableT, Ws, bs)


def _sc_pool_body(docs_hbm, table_hbm, out_hbm, slab_v, acc_v, sem):
    wid = lax.axis_index("s") * NC + lax.axis_index("c")
    base = wid * BPW

    pltpu.sync_copy(docs_hbm.at[:, pl.ds(base, BPW)], slab_v)

    zeros = jnp.zeros((16,), jnp.float32)

    def zero_body(bb, c2):
        for e in range(NP // 16):
            acc_v[bb, pl.ds(e * 16, 16)] = zeros
        return c2

    lax.fori_loop(0, BPW, zero_body, 0, unroll=8)

    cps = []
    for l in range(L):
        cps.append(
            pltpu.async_copy(
                table_hbm.at[slab_v.at[l]], acc_v, sem, add=True
            )
        )
        if l >= K:
            cps[l - K].wait()
    for l in range(L - K, L):
        cps[l].wait()

    pltpu.sync_copy(acc_v, out_hbm.at[pl.ds(base, BPW)])


def _sc_pool(docs, ptable):
    mesh = plsc.VectorSubcoreMesh(core_axis_name="c", subcore_axis_name="s")
    f = pl.kernel(
        _sc_pool_body,
        out_type=jax.ShapeDtypeStruct((B, NP), jnp.float32),
        mesh=mesh,
        scratch_types=[
            pltpu.VMEM((L, BPW), jnp.int32),
            pltpu.VMEM((BPW, NP), jnp.float32),
            pltpu.SemaphoreType.DMA,
        ],
        compiler_params=pltpu.CompilerParams(use_tc_tiling_on_sc=False),
    )
    return f(docs, ptable)


@jax.jit
def _run(docs, table, W, b):
    tableT = jnp.transpose(table)
    Ws = jnp.zeros((EMB, NP), jnp.float32).at[:, :NCLS].set(W.T) * (1.0 / L)
    bs = jnp.broadcast_to(
        jnp.zeros((NP,), jnp.float32).at[:NCLS].set(b) * (1.0 / L), (8, NP)
    )
    p2 = _project(tableT, Ws, bs)
    ptable = jnp.reshape(p2, (VP, NP))
    half = VB // 2
    o = docs & (VB - 1)
    r = o & (half - 1)
    h = o >> 9
    docs2 = (docs - o) + 2 * r + h
    scores = _sc_pool(docs2, ptable)
    return scores[:, :NCLS]


def kernel(docs, table, W, b):
    return _run(docs, table, W, b)


# speedup vs baseline: 1.6237x; 1.1019x over previous
import jax
import jax.numpy as jnp
from jax import lax
from jax.experimental import pallas as pl
from jax.experimental.pallas import tpu as pltpu
from jax.experimental.pallas import tpu_sc as plsc

L = 200
B = 4096
EMB = 64
NCLS = 50
VOCAB = 100000
NP = 64
CW = 32
QP = 128 // CW

NC = 2
NS = 16
NW = NC * NS
BPW = B // NW

K = 24

VB = 16384
PGRID = (VOCAB + VB - 1) // VB
VP = PGRID * VB


def _proj_body(tT_ref, ws_ref, bs_ref, o_ref):
    p = lax.dot_general(
        tT_ref[...], ws_ref[...],
        (((0,), (0,)), ((), ())),
        preferred_element_type=jnp.float32,
    )
    p = p + bs_ref[0:1, :]
    q = VB // QP
    o_ref[...] = jnp.concatenate(
        [p[i * q:(i + 1) * q, :] for i in range(QP)], axis=1
    )


def _project(tableT, Ws, bs):
    return pl.pallas_call(
        _proj_body,
        grid=(PGRID,),
        in_specs=[
            pl.BlockSpec((EMB, VB), lambda j: (0, j)),
            pl.BlockSpec((EMB, CW), lambda j: (0, 0)),
            pl.BlockSpec((8, CW), lambda j: (0, 0)),
        ],
        out_specs=pl.BlockSpec((VB // QP, 128), lambda j: (j, 0)),
        out_shape=jax.ShapeDtypeStruct((VP // QP, 128), jnp.float32),
    )(tableT, Ws, bs)


def _sc_pool_body(docs_hbm, table_hbm, out_hbm, slab_v, acc_v, sem):
    wid = lax.axis_index("s") * NC + lax.axis_index("c")
    base = wid * BPW

    pltpu.sync_copy(docs_hbm.at[:, pl.ds(base, BPW)], slab_v)

    zeros = jnp.zeros((16,), jnp.float32)

    def zero_body(bb, c2):
        for e in range(CW // 16):
            acc_v[bb, pl.ds(e * 16, 16)] = zeros
        return c2

    lax.fori_loop(0, BPW, zero_body, 0, unroll=8)

    cps = []
    for l in range(L):
        cps.append(
            pltpu.async_copy(
                table_hbm.at[slab_v.at[l]], acc_v, sem, add=True
            )
        )
        if l >= K:
            cps[l - K].wait()
    for l in range(L - K, L):
        cps[l].wait()

    pltpu.sync_copy(acc_v, out_hbm.at[pl.ds(base, BPW)])


def _sc_pool(docs, ptable):
    mesh = plsc.VectorSubcoreMesh(core_axis_name="c", subcore_axis_name="s")
    f = pl.kernel(
        _sc_pool_body,
        out_type=jax.ShapeDtypeStruct((B, CW), jnp.float32),
        mesh=mesh,
        scratch_types=[
            pltpu.VMEM((L, BPW), jnp.int32),
            pltpu.VMEM((BPW, CW), jnp.float32),
            pltpu.SemaphoreType.DMA,
        ],
        compiler_params=pltpu.CompilerParams(use_tc_tiling_on_sc=False),
    )
    return f(docs, ptable)


@jax.jit
def _run(docs, table, W, b):
    tableT = jnp.transpose(table)
    Ws = jnp.zeros((EMB, NP), jnp.float32).at[:, :NCLS].set(W.T) * (1.0 / L)
    bs8 = jnp.broadcast_to(
        jnp.zeros((NP,), jnp.float32).at[:NCLS].set(b) * (1.0 / L), (8, NP)
    )
    q = VB // QP
    o = docs % VB
    r = o % q
    g = o // q
    docs2 = (docs - o) + QP * r + g

    halves = []
    for h in range(2):
        p = _project(tableT, Ws[:, h * CW:(h + 1) * CW],
                     bs8[:, h * CW:(h + 1) * CW])
        ptable = jnp.reshape(p, (VP, CW))
        halves.append(_sc_pool(docs2, ptable))

    return jnp.concatenate(halves, axis=1)[:, :NCLS]


def kernel(docs, table, W, b):
    return _run(docs, table, W, b)


# speedup vs baseline: 1.9288x; 1.1880x over previous
import jax
import jax.numpy as jnp
from jax import lax
from jax.experimental import pallas as pl
from jax.experimental.pallas import tpu as pltpu
from jax.experimental.pallas import tpu_sc as plsc

L = 200
B = 4096
EMB = 64
NCLS = 50
VOCAB = 100000
NP = 64

NC = 2
NS = 16
NW = NC * NS
BPW = B // NW

K = 24

VB = 16384
PGRID = (VOCAB + VB - 1) // VB
VP = PGRID * VB


def _proj_body(tT_ref, ws_ref, bs_ref, o_ref):
    p = lax.dot_general(
        tT_ref[...], ws_ref[...],
        (((0,), (0,)), ((), ())),
        preferred_element_type=jnp.float32,
    )
    p = p + bs_ref[0:1, :]
    half = VB // 2
    o_ref[...] = jnp.concatenate([p[:half, :], p[half:, :]], axis=1)


def _project(tableT, Ws, bs):
    return pl.pallas_call(
        _proj_body,
        grid=(PGRID,),
        in_specs=[
            pl.BlockSpec((EMB, VB), lambda j: (0, j)),
            pl.BlockSpec((EMB, NP), lambda j: (0, 0)),
            pl.BlockSpec((8, NP), lambda j: (0, 0)),
        ],
        out_specs=pl.BlockSpec((VB // 2, 2 * NP), lambda j: (j, 0)),
        out_shape=jax.ShapeDtypeStruct((VP // 2, 2 * NP), jnp.float32),
    )(tableT, Ws, bs)


def _sc_pool_body(docs_hbm, table_hbm, out_hbm, slab_v, acc_v, sem):
    wid = lax.axis_index("s") * NC + lax.axis_index("c")
    base = wid * BPW

    pltpu.sync_copy(docs_hbm.at[:, pl.ds(base, BPW)], slab_v)

    zeros = jnp.zeros((16,), jnp.float32)

    def zero_body(bb, c2):
        for e in range(NP // 16):
            acc_v[bb, pl.ds(e * 16, 16)] = zeros
        return c2

    lax.fori_loop(0, BPW, zero_body, 0, unroll=8)

    cps = []
    for l in range(L):
        cps.append(
            pltpu.async_copy(
                table_hbm.at[slab_v.at[l]], acc_v, sem, add=True
            )
        )
        if l >= K:
            cps[l - K].wait()
    for l in range(L - K, L):
        cps[l].wait()

    pltpu.sync_copy(acc_v, out_hbm.at[pl.ds(base, BPW)])


def _sc_pool(docs, ptable):
    mesh = plsc.VectorSubcoreMesh(core_axis_name="c", subcore_axis_name="s")
    f = pl.kernel(
        _sc_pool_body,
        out_type=jax.ShapeDtypeStruct((B, NP), jnp.float32),
        mesh=mesh,
        scratch_types=[
            pltpu.VMEM((L, BPW), jnp.int32),
            pltpu.VMEM((BPW, NP), jnp.float32),
            pltpu.SemaphoreType.DMA,
        ],
        compiler_params=pltpu.CompilerParams(use_tc_tiling_on_sc=False),
    )
    return f(docs, ptable)


@jax.jit
def _run(docs, table, W, b):
    tableT = jnp.transpose(table)
    Ws = jnp.zeros((EMB, NP), jnp.float32).at[:, :NCLS].set(W.T) * (1.0 / L)
    bs = jnp.broadcast_to(
        jnp.zeros((NP,), jnp.float32).at[:NCLS].set(b) * (1.0 / L), (8, NP)
    )
    p2 = _project(tableT, Ws, bs)
    ptable = jnp.reshape(p2, (VP, NP))
    half = VB // 2
    o = docs % VB
    r = o % half
    h = o // half
    docs2 = (docs - o) + 2 * r + h
    scores = _sc_pool(docs2, ptable)
    return scores[:, :NCLS]


def kernel(docs, table, W, b):
    return _run(docs, table, W, b)
